# baseline (speedup 1.0000x reference)
import jax
import jax.numpy as jnp
from jax import lax
from jax.experimental import pallas as pl
from jax.experimental.pallas import tpu as pltpu
from jax.experimental.pallas import tpu_sc as plsc
import functools

N = 10000
E = 320000
D = 128

NC = 2
NS = 16
NW = NC * NS
E_PER_TILE = E // NW
CHUNK = 128
EP_PAD = 10240
N_CHUNKS = EP_PAD // CHUNK
NP = 10240
ROWS_PER_TILE = NP // NS
SUP = 16
NSUP = N_CHUNKS // SUP


def _fill_rows(buf, val):
    def step(t, carry):
        buf[t // 8, pl.ds((t % 8) * 16, 16)] = jnp.full((16,), val, jnp.float32)
        return carry
    lax.fori_loop(0, CHUNK * (D // 16), step, None)


def _sc_agg_body(with_cnt, *refs):
    if with_cnt:
        (x_hbm, src_hbm, dstf_hbm, acc_hbm, cnt_hbm, sh_acc, dst_v, sidx,
         rows_a, rows_b, sem_a, sem_b) = refs
    else:
        (x_hbm, src_hbm, dstf_hbm, acc_hbm, sh_acc, dst_v, sidx,
         rows_a, rows_b, sem_a, sem_b) = refs

    c = lax.axis_index("c")
    s = lax.axis_index("s")
    wid = c * NS + s
    row0 = s * ROWS_PER_TILE
    e0 = wid * EP_PAD

    pltpu.sync_copy(dstf_hbm.at[wid], dst_v)

    def zero_share():
        _fill_rows(rows_a, 0.0)
        for r in range(ROWS_PER_TILE // CHUNK):
            pltpu.sync_copy(rows_a, sh_acc.at[pl.ds(row0 + r * CHUNK, CHUNK)])

    if with_cnt:
        zero_share()
        _fill_rows(rows_b, 1.0)
        plsc.subcore_barrier()
        WIN = 8

        def cnt_step(i, carry):
            pltpu.async_copy(rows_b, sh_acc.at[dst_v.at[i]], sem_b, add=True)

            @pl.when(i >= WIN)
            def _():
                pltpu.make_async_copy(rows_b, sh_acc.at[pl.ds(0, CHUNK)],
                                      sem_b).wait()
            return carry

        lax.fori_loop(0, N_CHUNKS, cnt_step, None)

        def cnt_drain(i, carry):
            pltpu.make_async_copy(rows_b, sh_acc.at[pl.ds(0, CHUNK)],
                                  sem_b).wait()
            return carry

        lax.fori_loop(0, WIN, cnt_drain, None)
        plsc.subcore_barrier()
        pltpu.sync_copy(sh_acc.at[pl.ds(row0, ROWS_PER_TILE)],
                        cnt_hbm.at[c, pl.ds(row0, ROWS_PER_TILE)])
        plsc.subcore_barrier()

    zero_share()
    plsc.subcore_barrier()

    def gather(j, buf, sem):
        pltpu.async_copy(x_hbm.at[sidx.at[pl.ds(j * CHUNK, CHUNK)]], buf, sem)

    def gwait(buf, sem):
        pltpu.make_async_copy(x_hbm.at[pl.ds(0, CHUNK)], buf, sem).wait()

    def scat(j, buf):
        pltpu.sync_copy(buf, sh_acc.at[dst_v.at[j]], add=True)

    for sp in range(NSUP):
        c0 = sp * SUP
        pltpu.sync_copy(src_hbm.at[pl.ds(e0 + c0 * CHUNK, SUP * CHUNK)], sidx)
        gather(0, rows_a, sem_a)
        gather(1, rows_b, sem_b)

        def pair(t, carry):
            j0 = 2 * t
            gwait(rows_a, sem_a)
            scat(c0 + j0, rows_a)
            gather(j0 + 2, rows_a, sem_a)
            gwait(rows_b, sem_b)
            scat(c0 + j0 + 1, rows_b)
            gather(j0 + 3, rows_b, sem_b)
            return carry

        lax.fori_loop(0, (SUP - 2) // 2, pair, None)
        gwait(rows_a, sem_a)
        scat(c0 + SUP - 2, rows_a)
        gwait(rows_b, sem_b)
        scat(c0 + SUP - 1, rows_b)

    plsc.subcore_barrier()

    pltpu.sync_copy(sh_acc.at[pl.ds(row0, ROWS_PER_TILE)],
                    acc_hbm.at[c, pl.ds(row0, ROWS_PER_TILE)])


_sc_cache = {}


def _get_sc_agg(with_cnt):
    if with_cnt not in _sc_cache:
        if with_cnt:
            out_type = (jax.ShapeDtypeStruct((NC, NP, D), jnp.float32),
                        jax.ShapeDtypeStruct((NC, NP, D), jnp.float32))
        else:
            out_type = jax.ShapeDtypeStruct((NC, NP, D), jnp.float32)
        mesh = plsc.VectorSubcoreMesh(core_axis_name="c", subcore_axis_name="s")
        _sc_cache[with_cnt] = pl.kernel(
            functools.partial(_sc_agg_body, with_cnt),
            out_type=out_type,
            mesh=mesh,
            scratch_types=[
                pltpu.VMEM_SHARED((NP, D), jnp.float32),
                pltpu.VMEM((N_CHUNKS, CHUNK), jnp.int32),
                pltpu.VMEM((SUP * CHUNK,), jnp.int32),
                pltpu.VMEM((CHUNK, D), jnp.float32),
                pltpu.VMEM((CHUNK, D), jnp.float32),
                pltpu.SemaphoreType.DMA,
                pltpu.SemaphoreType.DMA,
            ],
        )
    return _sc_cache[with_cnt]


def _tc_layer_body(relu, x_ref, a_ref, c_ref, wn_ref, ws_ref, b_ref, o_ref):
    cnt = c_ref[0, :, 0:1] + c_ref[1, :, 0:1]
    scale = 1.0 / jnp.maximum(cnt, 1.0)
    neigh = (a_ref[0] + a_ref[1]) * scale
    dn = (((1,), (1,)), ((), ()))
    out = (lax.dot_general(x_ref[...], ws_ref[...], dn,
                           preferred_element_type=jnp.float32)
           + b_ref[...]
           + lax.dot_general(neigh, wn_ref[...], dn,
                             preferred_element_type=jnp.float32))
    if relu:
        out = jnp.maximum(out, 0.0)
    o_ref[...] = out


def _tc_layer(x, acc, cnt, Wn, Ws, b, relu):
    BN = 1000
    grid = (N // BN,)
    return pl.pallas_call(
        functools.partial(_tc_layer_body, relu),
        grid=grid,
        in_specs=[
            pl.BlockSpec((BN, D), lambda i: (i, 0)),
            pl.BlockSpec((NC, BN, D), lambda i: (0, i, 0)),
            pl.BlockSpec((NC, BN, D), lambda i: (0, i, 0)),
            pl.BlockSpec((D, D), lambda i: (0, 0)),
            pl.BlockSpec((D, D), lambda i: (0, 0)),
            pl.BlockSpec((1, D), lambda i: (0, 0)),
        ],
        out_specs=pl.BlockSpec((BN, D), lambda i: (i, 0)),
        out_shape=jax.ShapeDtypeStruct((N, D), jnp.float32),
    )(x, acc, cnt, Wn, Ws, b)


def kernel(x, edge_index, W_neigh1, W_self1, b_self1, W_neigh2, W_self2, b_self2):
    src = edge_index[0]
    dst = edge_index[1]
    pad = EP_PAD - E_PER_TILE
    src_p = jnp.pad(src.reshape(NW, E_PER_TILE), ((0, 0), (0, pad))).reshape(-1)
    dstf = jnp.pad(dst.reshape(NW, E_PER_TILE), ((0, 0), (0, pad)),
                   constant_values=NP - 1).reshape(NW, N_CHUNKS, CHUNK)
    acc1, cnt = _get_sc_agg(True)(x, src_p, dstf)
    h = _tc_layer(x, acc1, cnt, W_neigh1, W_self1, b_self1.reshape(1, D), True)
    acc2 = _get_sc_agg(False)(h, src_p, dstf)
    out = _tc_layer(h, acc2, cnt, W_neigh2, W_self2, b_self2.reshape(1, D), False)
    return out


# speedup vs baseline: 2.4784x; 2.4784x over previous
import jax
import jax.numpy as jnp
from jax import lax
from jax.experimental import pallas as pl
from jax.experimental.pallas import tpu as pltpu
from jax.experimental.pallas import tpu_sc as plsc
import functools

N = 10000
E = 320000
D = 128

NC = 2
NS = 16
NW = NC * NS
E_PER_TILE = E // NW
CHUNK = 80
EP_PAD = E_PER_TILE
N_CHUNKS = EP_PAD // CHUNK
NP = 10240
ROWS_PER_TILE = NP // NS
SUP = 25
NSUP = N_CHUNKS // SUP


def _fill_rows(buf, val):
    def step(t, carry):
        buf[t // 8, pl.ds((t % 8) * 16, 16)] = jnp.full((16,), val, jnp.float32)
        return carry
    lax.fori_loop(0, CHUNK * (D // 16), step, None)


def _sc_agg_body(with_cnt, *refs):
    if with_cnt:
        (x_hbm, src_hbm, dstf_hbm, acc_hbm, cnt_hbm, sh_acc, dst_v, sidx,
         rows_a, rows_b, sem_a, sem_b) = refs
    else:
        (x_hbm, src_hbm, dstf_hbm, acc_hbm, sh_acc, dst_v, sidx,
         rows_a, rows_b, sem_a, sem_b) = refs

    c = lax.axis_index("c")
    s = lax.axis_index("s")
    wid = c * NS + s
    row0 = s * ROWS_PER_TILE
    e0 = wid * EP_PAD

    pltpu.sync_copy(dstf_hbm.at[wid], dst_v)

    def zero_share():
        _fill_rows(rows_a, 0.0)
        for r in range(ROWS_PER_TILE // CHUNK):
            pltpu.sync_copy(rows_a, sh_acc.at[pl.ds(row0 + r * CHUNK, CHUNK)])

    if with_cnt:
        zero_share()
        _fill_rows(rows_b, 1.0)
        plsc.subcore_barrier()
        WIN = 8

        def cnt_step(i, carry):
            pltpu.async_copy(rows_b, sh_acc.at[dst_v.at[i]], sem_b, add=True)

            @pl.when(i >= WIN)
            def _():
                pltpu.make_async_copy(rows_b, sh_acc.at[pl.ds(0, CHUNK)],
                                      sem_b).wait()
            return carry

        lax.fori_loop(0, N_CHUNKS, cnt_step, None)

        def cnt_drain(i, carry):
            pltpu.make_async_copy(rows_b, sh_acc.at[pl.ds(0, CHUNK)],
                                  sem_b).wait()
            return carry

        lax.fori_loop(0, WIN, cnt_drain, None)
        plsc.subcore_barrier()
        pltpu.sync_copy(sh_acc.at[pl.ds(row0, ROWS_PER_TILE)],
                        cnt_hbm.at[c, pl.ds(row0, ROWS_PER_TILE)])
        plsc.subcore_barrier()

    zero_share()
    plsc.subcore_barrier()

    def gather(j, buf, sem):
        pltpu.async_copy(x_hbm.at[sidx.at[pl.ds(j * CHUNK, CHUNK)]], buf, sem)

    def gwait(buf, sem):
        pltpu.make_async_copy(x_hbm.at[pl.ds(0, CHUNK)], buf, sem).wait()

    def scat(j, buf):
        pltpu.sync_copy(buf, sh_acc.at[dst_v.at[j]], add=True)

    for sp in range(NSUP):
        c0 = sp * SUP
        pltpu.sync_copy(src_hbm.at[pl.ds(e0 + c0 * CHUNK, SUP * CHUNK)], sidx)
        gather(0, rows_a, sem_a)
        gather(1, rows_b, sem_b)

        def pair(t, carry):
            j0 = 2 * t
            gwait(rows_a, sem_a)
            scat(c0 + j0, rows_a)
            gather(j0 + 2, rows_a, sem_a)
            gwait(rows_b, sem_b)
            scat(c0 + j0 + 1, rows_b)
            gather(j0 + 3, rows_b, sem_b)
            return carry

        lax.fori_loop(0, (SUP - 3) // 2, pair, None)
        gwait(rows_a, sem_a)
        scat(c0 + SUP - 3, rows_a)
        gather(SUP - 1, rows_a, sem_a)
        gwait(rows_b, sem_b)
        scat(c0 + SUP - 2, rows_b)
        gwait(rows_a, sem_a)
        scat(c0 + SUP - 1, rows_a)

    plsc.subcore_barrier()

    pltpu.sync_copy(sh_acc.at[pl.ds(row0, ROWS_PER_TILE)],
                    acc_hbm.at[c, pl.ds(row0, ROWS_PER_TILE)])


_sc_cache = {}


def _get_sc_agg(with_cnt):
    if with_cnt not in _sc_cache:
        if with_cnt:
            out_type = (jax.ShapeDtypeStruct((NC, NP, D), jnp.float32),
                        jax.ShapeDtypeStruct((NC, NP, D), jnp.float32))
        else:
            out_type = jax.ShapeDtypeStruct((NC, NP, D), jnp.float32)
        mesh = plsc.VectorSubcoreMesh(core_axis_name="c", subcore_axis_name="s")
        _sc_cache[with_cnt] = pl.kernel(
            functools.partial(_sc_agg_body, with_cnt),
            out_type=out_type,
            mesh=mesh,
            scratch_types=[
                pltpu.VMEM_SHARED((NP, D), jnp.float32),
                pltpu.VMEM((N_CHUNKS, CHUNK), jnp.int32),
                pltpu.VMEM((SUP * CHUNK,), jnp.int32),
                pltpu.VMEM((CHUNK, D), jnp.float32),
                pltpu.VMEM((CHUNK, D), jnp.float32),
                pltpu.SemaphoreType.DMA,
                pltpu.SemaphoreType.DMA,
            ],
        )
    return _sc_cache[with_cnt]


def _tc_layer_body(relu, x_ref, a_ref, c_ref, wn_ref, ws_ref, b_ref, o_ref):
    cnt = c_ref[0, :, 0:1] + c_ref[1, :, 0:1]
    scale = 1.0 / jnp.maximum(cnt, 1.0)
    neigh = (a_ref[0] + a_ref[1]) * scale
    dn = (((1,), (1,)), ((), ()))
    out = (lax.dot_general(x_ref[...], ws_ref[...], dn,
                           preferred_element_type=jnp.float32)
           + b_ref[...]
           + lax.dot_general(neigh, wn_ref[...], dn,
                             preferred_element_type=jnp.float32))
    if relu:
        out = jnp.maximum(out, 0.0)
    o_ref[...] = out


def _tc_layer(x, acc, cnt, Wn, Ws, b, relu):
    BN = 1000
    grid = (N // BN,)
    return pl.pallas_call(
        functools.partial(_tc_layer_body, relu),
        grid=grid,
        in_specs=[
            pl.BlockSpec((BN, D), lambda i: (i, 0)),
            pl.BlockSpec((NC, BN, D), lambda i: (0, i, 0)),
            pl.BlockSpec((NC, BN, D), lambda i: (0, i, 0)),
            pl.BlockSpec((D, D), lambda i: (0, 0)),
            pl.BlockSpec((D, D), lambda i: (0, 0)),
            pl.BlockSpec((1, D), lambda i: (0, 0)),
        ],
        out_specs=pl.BlockSpec((BN, D), lambda i: (i, 0)),
        out_shape=jax.ShapeDtypeStruct((N, D), jnp.float32),
    )(x, acc, cnt, Wn, Ws, b)


def kernel(x, edge_index, W_neigh1, W_self1, b_self1, W_neigh2, W_self2, b_self2):
    src = edge_index[0]
    dst = edge_index[1]
    src_p = src
    dstf = dst.reshape(NW, N_CHUNKS, CHUNK)
    acc1, cnt = _get_sc_agg(True)(x, src_p, dstf)
    h = _tc_layer(x, acc1, cnt, W_neigh1, W_self1, b_self1.reshape(1, D), True)
    acc2 = _get_sc_agg(False)(h, src_p, dstf)
    out = _tc_layer(h, acc2, cnt, W_neigh2, W_self2, b_self2.reshape(1, D), False)
    return out


# speedup vs baseline: 2.4852x; 1.0027x over previous
import jax
import jax.numpy as jnp
from jax import lax
from jax.experimental import pallas as pl
from jax.experimental.pallas import tpu as pltpu
from jax.experimental.pallas import tpu_sc as plsc
import functools

N = 10000
E = 320000
D = 128

NC = 2
NS = 16
NW = NC * NS
E_PER_TILE = E // NW
CHUNK = 80
EP_PAD = E_PER_TILE
N_CHUNKS = EP_PAD // CHUNK
NP = 10240
ROWS_PER_TILE = NP // NS
SUP = 25
NSUP = N_CHUNKS // SUP


def _fill_rows(buf, val):
    def step(t, carry):
        buf[t // 8, pl.ds((t % 8) * 16, 16)] = jnp.full((16,), val, jnp.float32)
        return carry
    lax.fori_loop(0, CHUNK * (D // 16), step, None)


def _sc_agg_body(with_cnt, *refs):
    if with_cnt:
        (x_hbm, src_hbm, dstf_hbm, acc_hbm, cnt_hbm, sh_acc, dst_v, sidx,
         rows_a, rows_b, sem_a, sem_b) = refs
    else:
        (x_hbm, src_hbm, dstf_hbm, acc_hbm, sh_acc, dst_v, sidx,
         rows_a, rows_b, sem_a, sem_b) = refs

    c = lax.axis_index("c")
    s = lax.axis_index("s")
    wid = c * NS + s
    row0 = s * ROWS_PER_TILE
    e0 = wid * EP_PAD

    pltpu.sync_copy(dstf_hbm.at[wid], dst_v)

    def zero_share():
        _fill_rows(rows_a, 0.0)
        for r in range(ROWS_PER_TILE // CHUNK):
            pltpu.async_copy(rows_a, sh_acc.at[pl.ds(row0 + r * CHUNK, CHUNK)],
                             sem_a)
        for r in range(ROWS_PER_TILE // CHUNK):
            pltpu.make_async_copy(rows_a, sh_acc.at[pl.ds(0, CHUNK)],
                                  sem_a).wait()

    if with_cnt:
        zero_share()
        _fill_rows(rows_b, 1.0)
        plsc.subcore_barrier()
        WIN = 8

        def cnt_step(i, carry):
            pltpu.async_copy(rows_b, sh_acc.at[dst_v.at[i]], sem_b, add=True)

            @pl.when(i >= WIN)
            def _():
                pltpu.make_async_copy(rows_b, sh_acc.at[pl.ds(0, CHUNK)],
                                      sem_b).wait()
            return carry

        lax.fori_loop(0, N_CHUNKS, cnt_step, None)

        def cnt_drain(i, carry):
            pltpu.make_async_copy(rows_b, sh_acc.at[pl.ds(0, CHUNK)],
                                  sem_b).wait()
            return carry

        lax.fori_loop(0, WIN, cnt_drain, None)
        plsc.subcore_barrier()
        pltpu.sync_copy(sh_acc.at[pl.ds(row0, ROWS_PER_TILE)],
                        cnt_hbm.at[c, pl.ds(row0, ROWS_PER_TILE)])
        plsc.subcore_barrier()

    zero_share()
    plsc.subcore_barrier()

    def gather(j, buf, sem):
        pltpu.async_copy(x_hbm.at[sidx.at[pl.ds(j * CHUNK, CHUNK)]], buf, sem)

    def gwait(buf, sem):
        pltpu.make_async_copy(x_hbm.at[pl.ds(0, CHUNK)], buf, sem).wait()

    def scat(j, buf):
        pltpu.sync_copy(buf, sh_acc.at[dst_v.at[j]], add=True)

    for sp in range(NSUP):
        c0 = sp * SUP
        pltpu.sync_copy(src_hbm.at[pl.ds(e0 + c0 * CHUNK, SUP * CHUNK)], sidx)
        gather(0, rows_a, sem_a)
        gather(1, rows_b, sem_b)

        def pair(t, carry):
            j0 = 2 * t
            gwait(rows_a, sem_a)
            scat(c0 + j0, rows_a)
            gather(j0 + 2, rows_a, sem_a)
            gwait(rows_b, sem_b)
            scat(c0 + j0 + 1, rows_b)
            gather(j0 + 3, rows_b, sem_b)
            return carry

        lax.fori_loop(0, (SUP - 3) // 2, pair, None)
        gwait(rows_a, sem_a)
        scat(c0 + SUP - 3, rows_a)
        gather(SUP - 1, rows_a, sem_a)
        gwait(rows_b, sem_b)
        scat(c0 + SUP - 2, rows_b)
        gwait(rows_a, sem_a)
        scat(c0 + SUP - 1, rows_a)

    plsc.subcore_barrier()

    pltpu.sync_copy(sh_acc.at[pl.ds(row0, ROWS_PER_TILE)],
                    acc_hbm.at[c, pl.ds(row0, ROWS_PER_TILE)])


_sc_cache = {}


def _get_sc_agg(with_cnt):
    if with_cnt not in _sc_cache:
        if with_cnt:
            out_type = (jax.ShapeDtypeStruct((NC, NP, D), jnp.float32),
                        jax.ShapeDtypeStruct((NC, NP, D), jnp.float32))
        else:
            out_type = jax.ShapeDtypeStruct((NC, NP, D), jnp.float32)
        mesh = plsc.VectorSubcoreMesh(core_axis_name="c", subcore_axis_name="s")
        _sc_cache[with_cnt] = pl.kernel(
            functools.partial(_sc_agg_body, with_cnt),
            out_type=out_type,
            mesh=mesh,
            scratch_types=[
                pltpu.VMEM_SHARED((NP, D), jnp.float32),
                pltpu.VMEM((N_CHUNKS, CHUNK), jnp.int32),
                pltpu.VMEM((SUP * CHUNK,), jnp.int32),
                pltpu.VMEM((CHUNK, D), jnp.float32),
                pltpu.VMEM((CHUNK, D), jnp.float32),
                pltpu.SemaphoreType.DMA,
                pltpu.SemaphoreType.DMA,
            ],
        )
    return _sc_cache[with_cnt]


def _tc_layer_body(relu, x_ref, a_ref, c_ref, wn_ref, ws_ref, b_ref, o_ref):
    cnt = c_ref[0, :, 0:1] + c_ref[1, :, 0:1]
    scale = 1.0 / jnp.maximum(cnt, 1.0)
    neigh = (a_ref[0] + a_ref[1]) * scale
    dn = (((1,), (1,)), ((), ()))
    out = (lax.dot_general(x_ref[...], ws_ref[...], dn,
                           preferred_element_type=jnp.float32)
           + b_ref[...]
           + lax.dot_general(neigh, wn_ref[...], dn,
                             preferred_element_type=jnp.float32))
    if relu:
        out = jnp.maximum(out, 0.0)
    o_ref[...] = out


def _tc_layer(x, acc, cnt, Wn, Ws, b, relu):
    BN = 1000
    grid = (N // BN,)
    return pl.pallas_call(
        functools.partial(_tc_layer_body, relu),
        grid=grid,
        in_specs=[
            pl.BlockSpec((BN, D), lambda i: (i, 0)),
            pl.BlockSpec((NC, BN, D), lambda i: (0, i, 0)),
            pl.BlockSpec((NC, BN, D), lambda i: (0, i, 0)),
            pl.BlockSpec((D, D), lambda i: (0, 0)),
            pl.BlockSpec((D, D), lambda i: (0, 0)),
            pl.BlockSpec((1, D), lambda i: (0, 0)),
        ],
        out_specs=pl.BlockSpec((BN, D), lambda i: (i, 0)),
        out_shape=jax.ShapeDtypeStruct((N, D), jnp.float32),
    )(x, acc, cnt, Wn, Ws, b)


def kernel(x, edge_index, W_neigh1, W_self1, b_self1, W_neigh2, W_self2, b_self2):
    src = edge_index[0]
    dst = edge_index[1]
    src_p = src
    dstf = dst.reshape(NW, N_CHUNKS, CHUNK)
    acc1, cnt = _get_sc_agg(True)(x, src_p, dstf)
    h = _tc_layer(x, acc1, cnt, W_neigh1, W_self1, b_self1.reshape(1, D), True)
    acc2 = _get_sc_agg(False)(h, src_p, dstf)
    out = _tc_layer(h, acc2, cnt, W_neigh2, W_self2, b_self2.reshape(1, D), False)
    return out
